# baseline (speedup 1.0000x reference)
import numpy as np
import jax
import jax.numpy as jnp
from jax import lax
from jax.experimental import pallas as pl
from jax.experimental.pallas import tpu as pltpu
from jax.experimental.pallas import tpu_sc as plsc

_N_LEVELS = 16
_BASE_RES = 16
_MAX_RES = 2048
_T = 2 ** 19
_F = 2
_N_POINTS = 262144
_growth = np.exp((np.log(_MAX_RES) - np.log(_BASE_RES)) / (_N_LEVELS - 1))
_RES = [int(np.floor(_BASE_RES * _growth ** l)) for l in range(_N_LEVELS)]
_P1 = np.uint32(2654435761).astype(np.int32)
_P2 = np.int32(805459861)
_MASK = np.int32(_T - 1)

_NC, _NS = 2, 16
_NW = _NC * _NS
_PER_W = _N_POINTS // _NW
_C = 1024
_NCHUNK = _PER_W // _C
_L = 16
_OUTW = 2 * _N_LEVELS
_OUT_SZ = _N_POINTS * _OUTW


def _body(x_hbm, tbl_hbm, out_hbm, xv, frv, idxv, rowsv, outv, cidxv,
          sem_g, sem_o):
    wid = lax.axis_index("s") * _NC + lax.axis_index("c")
    iota = lax.iota(jnp.int32, _L)

    def chunk_body(chunk, carry):
        base = wid * _PER_W + chunk * _C

        def i_body(i, c):
            s = pl.ds(i * _L, _L)
            k = base + i * _L + iota
            k3 = k + k + k
            for d in range(3):
                cidxv[d][s] = k3 + d
            return c

        lax.fori_loop(0, _C // _L, i_body, 0)

        xg = [pltpu.async_copy(x_hbm.at[cidxv[d]], xv[d], sem_g)
              for d in range(3)]
        for g in xg:
            g.wait()

        out_copies = []
        for l in range(_N_LEVELS):
            res = _RES[l]
            dense = (res + 1) ** 3 <= _T
            resf = jnp.float32(res)
            resi = jnp.int32(res)
            ofs2 = jnp.int32(2 * l * _T)

            def a_body(i, c, res=res, dense=dense, resf=resf, resi=resi,
                       ofs2=ofs2):
                s = pl.ds(i * _L, _L)
                lo, hi = [], []
                for d in range(3):
                    p = xv[d][s] * resf
                    ii = p.astype(jnp.int32)
                    frv[d][s] = p - ii.astype(jnp.float32)
                    lo.append(ii)
                    hi.append(jnp.minimum(ii + 1, resi))
                if dense:
                    r1 = jnp.int32(res + 1)
                    r2 = jnp.int32((res + 1) * (res + 1))
                    t1 = [lo[1] * r1, hi[1] * r1]
                    t2 = [lo[2] * r2, hi[2] * r2]
                else:
                    t1 = [lo[1] * _P1, hi[1] * _P1]
                    t2 = [lo[2] * _P2, hi[2] * _P2]
                for corner in range(8):
                    b0 = corner & 1
                    b1 = (corner >> 1) & 1
                    b2 = (corner >> 2) & 1
                    if dense:
                        row = [lo[0], hi[0]][b0] + t1[b1] + t2[b2]
                    else:
                        row = ([lo[0], hi[0]][b0] ^ t1[b1] ^ t2[b2]) & _MASK
                    e0 = row + row + ofs2
                    idxv[2 * corner][s] = e0
                    idxv[2 * corner + 1][s] = e0 + 1
                return c

            lax.fori_loop(0, _C // _L, a_body, 0)

            gathers = [pltpu.async_copy(tbl_hbm.at[idxv[j]], rowsv[j], sem_g)
                       for j in range(16)]
            for g in gathers:
                g.wait()

            def b_body(i, c, l=l):
                s = pl.ds(i * _L, _L)
                fr = [frv[d][s] for d in range(3)]
                om = [1.0 - f for f in fr]
                acc0 = jnp.zeros((_L,), jnp.float32)
                acc1 = jnp.zeros((_L,), jnp.float32)
                for corner in range(8):
                    b0 = corner & 1
                    b1 = (corner >> 1) & 1
                    b2 = (corner >> 2) & 1
                    w = ([om[0], fr[0]][b0] * [om[1], fr[1]][b1]) \
                        * [om[2], fr[2]][b2]
                    acc0 = acc0 + w * rowsv[2 * corner][s]
                    acc1 = acc1 + w * rowsv[2 * corner + 1][s]
                outv[2 * l][s] = acc0
                outv[2 * l + 1][s] = acc1
                return c

            lax.fori_loop(0, _C // _L, b_body, 0)

            for f in range(2):
                p = 2 * l + f
                out_copies.append(pltpu.async_copy(
                    outv[p], out_hbm.at[pl.ds(p * _N_POINTS + base, _C)],
                    sem_o))

        for oc in out_copies:
            oc.wait()
        return carry

    lax.fori_loop(0, _NCHUNK, chunk_body, 0)


def kernel(in_tensor, table):
    x_flat = in_tensor.reshape(_N_POINTS * 3)
    tbl = table.reshape(_N_LEVELS * _T * _F)
    mesh = plsc.VectorSubcoreMesh(core_axis_name="c", subcore_axis_name="s")
    f = pl.kernel(
        _body,
        out_type=jax.ShapeDtypeStruct((_OUT_SZ,), jnp.float32),
        mesh=mesh,
        scratch_types=[
            [pltpu.VMEM((_C,), jnp.float32) for _ in range(3)],
            [pltpu.VMEM((_C,), jnp.float32) for _ in range(3)],
            [pltpu.VMEM((_C,), jnp.int32) for _ in range(16)],
            [pltpu.VMEM((_C,), jnp.float32) for _ in range(16)],
            [pltpu.VMEM((_C,), jnp.float32) for _ in range(32)],
            [pltpu.VMEM((_C,), jnp.int32) for _ in range(3)],
            pltpu.SemaphoreType.DMA,
            pltpu.SemaphoreType.DMA,
        ],
    )
    out = f(x_flat, tbl)
    return _tc_interleave(out)


_BN = 2048


def _tc_interleave_body(*refs):
    in_refs, out_ref = refs[:_OUTW], refs[_OUTW]
    for p in range(_OUTW):
        out_ref[:, p] = in_refs[p][...]


def _tc_interleave(flat):
    nblk = _N_POINTS // _BN
    return pl.pallas_call(
        _tc_interleave_body,
        grid=(nblk,),
        in_specs=[
            pl.BlockSpec((_BN,), lambda i, p=p: (p * nblk + i,))
            for p in range(_OUTW)
        ],
        out_specs=pl.BlockSpec((_BN, _OUTW), lambda i: (i, 0)),
        out_shape=jax.ShapeDtypeStruct((_N_POINTS, _OUTW), jnp.float32),
    )(*([flat] * _OUTW))


# speedup vs baseline: 4.6487x; 4.6487x over previous
import numpy as np
import jax
import jax.numpy as jnp
from jax import lax
from jax.experimental import pallas as pl
from jax.experimental.pallas import tpu as pltpu
from jax.experimental.pallas import tpu_sc as plsc

_N_LEVELS = 16
_BASE_RES = 16
_MAX_RES = 2048
_T = 2 ** 19
_F = 2
_N_POINTS = 262144
_growth = np.exp((np.log(_MAX_RES) - np.log(_BASE_RES)) / (_N_LEVELS - 1))
_RES = [int(np.floor(_BASE_RES * _growth ** l)) for l in range(_N_LEVELS)]
_P1 = np.uint32(2654435761).astype(np.int32)
_P2 = np.int32(805459861)
_MASK = np.int32(_T - 1)

_NC, _NS = 2, 16
_NW = _NC * _NS
_PER_W = _N_POINTS // _NW
_C = 1024
_NCHUNK = _PER_W // _C
_L = 16
_OUTW = 2 * _N_LEVELS
_OUT_SZ = _N_POINTS * _OUTW


def _body(x_hbm, tbl_hbm, out_hbm, xv, frv, idxv, rowsv, outv, cidxv,
          sem_g, sem_o):
    wid = lax.axis_index("s") * _NC + lax.axis_index("c")
    iota = lax.iota(jnp.int32, _L)

    def chunk_body(chunk, carry):
        base = wid * _PER_W + chunk * _C

        def i_body(i, c):
            s = pl.ds(i * _L, _L)
            k = base + i * _L + iota
            k3 = k + k + k
            for d in range(3):
                cidxv[d][s] = k3 + d
            return c

        lax.fori_loop(0, _C // _L, i_body, 0)

        xg = [pltpu.async_copy(x_hbm.at[cidxv[d]], xv[d], sem_g)
              for d in range(3)]
        for g in xg:
            g.wait()

        out_copies = []
        for l in range(_N_LEVELS):
            res = _RES[l]
            dense = (res + 1) ** 3 <= _T
            resf = jnp.float32(res)
            resi = jnp.int32(res)
            ofs2 = jnp.int32(2 * l * _T)

            def a_body(i, c, res=res, dense=dense, resf=resf, resi=resi,
                       ofs2=ofs2):
                s = pl.ds(i * _L, _L)
                lo, hi = [], []
                for d in range(3):
                    p = xv[d][s] * resf
                    ii = p.astype(jnp.int32)
                    frv[d][s] = p - ii.astype(jnp.float32)
                    lo.append(ii)
                    hi.append(jnp.minimum(ii + 1, resi))
                if dense:
                    r1 = jnp.int32(res + 1)
                    r2 = jnp.int32((res + 1) * (res + 1))
                    t1 = [lo[1] * r1, hi[1] * r1]
                    t2 = [lo[2] * r2, hi[2] * r2]
                else:
                    t1 = [lo[1] * _P1, hi[1] * _P1]
                    t2 = [lo[2] * _P2, hi[2] * _P2]
                for corner in range(8):
                    b0 = corner & 1
                    b1 = (corner >> 1) & 1
                    b2 = (corner >> 2) & 1
                    if dense:
                        row = [lo[0], hi[0]][b0] + t1[b1] + t2[b2]
                    else:
                        row = ([lo[0], hi[0]][b0] ^ t1[b1] ^ t2[b2]) & _MASK
                    tlo = row & jnp.int32(127)
                    e0 = ofs2 + (row - tlo) + (row - tlo) + tlo
                    idxv[2 * corner][s] = e0
                    idxv[2 * corner + 1][s] = e0 + 128
                return c

            lax.fori_loop(0, _C // _L, a_body, 0)

            gathers = [pltpu.async_copy(tbl_hbm.at[idxv[j]], rowsv[j], sem_g)
                       for j in range(16)]
            for g in gathers:
                g.wait()

            def b_body(i, c, l=l):
                s = pl.ds(i * _L, _L)
                fr = [frv[d][s] for d in range(3)]
                om = [1.0 - f for f in fr]
                acc0 = jnp.zeros((_L,), jnp.float32)
                acc1 = jnp.zeros((_L,), jnp.float32)
                for corner in range(8):
                    b0 = corner & 1
                    b1 = (corner >> 1) & 1
                    b2 = (corner >> 2) & 1
                    w = ([om[0], fr[0]][b0] * [om[1], fr[1]][b1]) \
                        * [om[2], fr[2]][b2]
                    acc0 = acc0 + w * rowsv[2 * corner][s]
                    acc1 = acc1 + w * rowsv[2 * corner + 1][s]
                outv[2 * l][s] = acc0
                outv[2 * l + 1][s] = acc1
                return c

            lax.fori_loop(0, _C // _L, b_body, 0)

            for f in range(2):
                p = 2 * l + f
                fblk, fin = p // 8, p % 8
                for blk in range(_C // 128):
                    dst = (fblk * (8 * _N_POINTS) + fin * 128
                           + (base // 128 + blk) * 1024)
                    out_copies.append(pltpu.async_copy(
                        outv[p].at[pl.ds(blk * 128, 128)],
                        out_hbm.at[pl.ds(dst, 128)], sem_o))

        for oc in out_copies:
            oc.wait()
        return carry

    lax.fori_loop(0, _NCHUNK, chunk_body, 0)


def kernel(in_tensor, table):
    x_flat = in_tensor.reshape(_N_POINTS * 3)
    tbl = (table.reshape(_N_LEVELS, _T // 128, 128, _F)
           .transpose(0, 1, 3, 2).reshape(_N_LEVELS * _T * _F))
    mesh = plsc.VectorSubcoreMesh(core_axis_name="c", subcore_axis_name="s")
    f = pl.kernel(
        _body,
        out_type=jax.ShapeDtypeStruct((_OUT_SZ,), jnp.float32),
        mesh=mesh,
        scratch_types=[
            [pltpu.VMEM((_C,), jnp.float32) for _ in range(3)],
            [pltpu.VMEM((_C,), jnp.float32) for _ in range(3)],
            [pltpu.VMEM((_C,), jnp.int32) for _ in range(16)],
            [pltpu.VMEM((_C,), jnp.float32) for _ in range(16)],
            [pltpu.VMEM((_C,), jnp.float32) for _ in range(32)],
            [pltpu.VMEM((_C,), jnp.int32) for _ in range(3)],
            pltpu.SemaphoreType.DMA,
            pltpu.SemaphoreType.DMA,
        ],
    )
    out = f(x_flat, tbl)
    return (out.reshape(4, _N_POINTS // 128, 8, 128)
            .transpose(1, 3, 0, 2).reshape(_N_POINTS, _OUTW))


# speedup vs baseline: 5.2375x; 1.1267x over previous
import numpy as np
import jax
import jax.numpy as jnp
from jax import lax
from jax.experimental import pallas as pl
from jax.experimental.pallas import tpu as pltpu
from jax.experimental.pallas import tpu_sc as plsc

_N_LEVELS = 16
_BASE_RES = 16
_MAX_RES = 2048
_T = 2 ** 19
_F = 2
_N_POINTS = 262144
_growth = np.exp((np.log(_MAX_RES) - np.log(_BASE_RES)) / (_N_LEVELS - 1))
_RES = [int(np.floor(_BASE_RES * _growth ** l)) for l in range(_N_LEVELS)]
_P1 = np.uint32(2654435761).astype(np.int32)
_P2 = np.int32(805459861)
_MASK = np.int32(_T - 1)

_NC, _NS = 2, 16
_NW = _NC * _NS
_PER_W = _N_POINTS // _NW
_C = 1024
_NCHUNK = _PER_W // _C
_L = 16
_OUTW = 2 * _N_LEVELS
_OUT_SZ = _N_POINTS * _OUTW


def _body(x_hbm, tbl_hbm, out_hbm, xv, frv, idxv, rowsv, outv, cidxv,
          sem_a, sem_b, sem_o):
    wid = lax.axis_index("s") * _NC + lax.axis_index("c")
    iota = lax.iota(jnp.int32, _L)
    sem_g = [sem_a, sem_b]

    def chunk_body(chunk, carry):
        base = wid * _PER_W + chunk * _C

        def i_body(i, c):
            s = pl.ds(i * _L, _L)
            k = base + i * _L + iota
            k3 = k + k + k
            for d in range(3):
                cidxv[d][s] = k3 + d
            return c

        lax.fori_loop(0, _C // _L, i_body, 0)

        xg = [pltpu.async_copy(x_hbm.at[cidxv[d]], xv[d], sem_a)
              for d in range(3)]
        for g in xg:
            g.wait()

        def index_pass_and_fire(l, st):
            res = _RES[l]
            dense = (res + 1) ** 3 <= _T
            resf = jnp.float32(res)
            resi = jnp.int32(res)
            ofs2 = jnp.int32(2 * l * _T)

            def a_body(i, c):
                s = pl.ds(i * _L, _L)
                lo, hi = [], []
                for d in range(3):
                    p = xv[d][s] * resf
                    ii = p.astype(jnp.int32)
                    frv[st][d][s] = p - ii.astype(jnp.float32)
                    lo.append(ii)
                    hi.append(jnp.minimum(ii + 1, resi))
                if dense:
                    r1 = jnp.int32(res + 1)
                    r2 = jnp.int32((res + 1) * (res + 1))
                    t1 = [lo[1] * r1, hi[1] * r1]
                    t2 = [lo[2] * r2, hi[2] * r2]
                else:
                    t1 = [lo[1] * _P1, hi[1] * _P1]
                    t2 = [lo[2] * _P2, hi[2] * _P2]
                for corner in range(8):
                    b0 = corner & 1
                    b1 = (corner >> 1) & 1
                    b2 = (corner >> 2) & 1
                    if dense:
                        row = [lo[0], hi[0]][b0] + t1[b1] + t2[b2]
                    else:
                        row = ([lo[0], hi[0]][b0] ^ t1[b1] ^ t2[b2]) & _MASK
                    tlo = row & jnp.int32(127)
                    e0 = ofs2 + (row - tlo) + (row - tlo) + tlo
                    idxv[st][2 * corner][s] = e0
                    idxv[st][2 * corner + 1][s] = e0 + 128
                return c

            lax.fori_loop(0, _C // _L, a_body, 0)
            return [pltpu.async_copy(tbl_hbm.at[idxv[st][j]], rowsv[st][j],
                                     sem_g[st])
                    for j in range(16)]

        out_copies = []
        gathers = index_pass_and_fire(0, 0)
        for l in range(_N_LEVELS):
            st = l % 2
            if l + 1 < _N_LEVELS:
                next_gathers = index_pass_and_fire(l + 1, (l + 1) % 2)
            for g in gathers:
                g.wait()

            def b_body(i, c, l=l, st=st):
                s = pl.ds(i * _L, _L)
                fr = [frv[st][d][s] for d in range(3)]
                om = [1.0 - f for f in fr]
                acc0 = jnp.zeros((_L,), jnp.float32)
                acc1 = jnp.zeros((_L,), jnp.float32)
                for corner in range(8):
                    b0 = corner & 1
                    b1 = (corner >> 1) & 1
                    b2 = (corner >> 2) & 1
                    w = ([om[0], fr[0]][b0] * [om[1], fr[1]][b1]) \
                        * [om[2], fr[2]][b2]
                    acc0 = acc0 + w * rowsv[st][2 * corner][s]
                    acc1 = acc1 + w * rowsv[st][2 * corner + 1][s]
                outv[2 * l][s] = acc0
                outv[2 * l + 1][s] = acc1
                return c

            lax.fori_loop(0, _C // _L, b_body, 0)

            for f in range(2):
                p = 2 * l + f
                fblk, fin = p // 8, p % 8
                for blk in range(_C // 128):
                    dst = (fblk * (8 * _N_POINTS) + fin * 128
                           + (base // 128 + blk) * 1024)
                    out_copies.append(pltpu.async_copy(
                        outv[p].at[pl.ds(blk * 128, 128)],
                        out_hbm.at[pl.ds(dst, 128)], sem_o))
            if l + 1 < _N_LEVELS:
                gathers = next_gathers

        for oc in out_copies:
            oc.wait()
        return carry

    lax.fori_loop(0, _NCHUNK, chunk_body, 0)


def kernel(in_tensor, table):
    x_flat = in_tensor.reshape(_N_POINTS * 3)
    tbl = (table.reshape(_N_LEVELS, _T // 128, 128, _F)
           .transpose(0, 1, 3, 2).reshape(_N_LEVELS * _T * _F))
    mesh = plsc.VectorSubcoreMesh(core_axis_name="c", subcore_axis_name="s")
    f = pl.kernel(
        _body,
        out_type=jax.ShapeDtypeStruct((_OUT_SZ,), jnp.float32),
        mesh=mesh,
        scratch_types=[
            [pltpu.VMEM((_C,), jnp.float32) for _ in range(3)],
            [[pltpu.VMEM((_C,), jnp.float32) for _ in range(3)]
             for _ in range(2)],
            [[pltpu.VMEM((_C,), jnp.int32) for _ in range(16)]
             for _ in range(2)],
            [[pltpu.VMEM((_C,), jnp.float32) for _ in range(16)]
             for _ in range(2)],
            [pltpu.VMEM((_C,), jnp.float32) for _ in range(32)],
            [pltpu.VMEM((_C,), jnp.int32) for _ in range(3)],
            pltpu.SemaphoreType.DMA,
            pltpu.SemaphoreType.DMA,
            pltpu.SemaphoreType.DMA,
        ],
    )
    out = f(x_flat, tbl)
    return (out.reshape(4, _N_POINTS // 128, 8, 128)
            .transpose(1, 3, 0, 2).reshape(_N_POINTS, _OUTW))
